# baseline (speedup 1.0000x reference)
import functools

import jax
import jax.numpy as jnp
from jax import lax
from jax.experimental import pallas as pl
from jax.experimental.pallas import tpu as pltpu
from jax.experimental.pallas import tpu_sc as plsc

_B = 16384
_D = 32
_V = 1000000
_RV = 1024
_NT = (_V + 127) // 128
_NTF = _V // 128
_WC = 16
_WE = _WC * 128
_CAP = 4096 + 32
_STRIDE = 128
_ACC = 144
_CHI = 2048

_info = plsc.get_sparse_core_info()
_NC, _NS = _info.num_cores, _info.num_subcores
_NW = _NC * _NS
_BPW = _B // _NW

_mesh = plsc.VectorSubcoreMesh(core_axis_name="c", subcore_axis_name="s")
_cp = pltpu.CompilerParams(use_tc_tiling_on_sc=True, needs_layout_passes=False)


def _make_k1():
    @functools.partial(
        pl.kernel,
        mesh=_mesh,
        compiler_params=_cp,
        out_type=jax.ShapeDtypeStruct((_B + 16 * _NW, 128), jnp.float32),
        scratch_types=[
            pltpu.VMEM((_CHI,), jnp.int32),
            pltpu.VMEM((_CAP,), jnp.int32),
            pltpu.VMEM((_CAP,), jnp.int32),
            pltpu.VMEM((48,), jnp.int32),
            pltpu.VMEM((48,), jnp.int32),
            pltpu.VMEM((_D, _WE), jnp.float32),
            pltpu.VMEM((_D, 64), jnp.float32),
            pltpu.VMEM((_D, 64), jnp.float32),
            pltpu.VMEM((_ACC, _STRIDE), jnp.float32),
            pltpu.VMEM((_ACC,), jnp.int32),
            pltpu.SemaphoreType.DMA,
            pltpu.SemaphoreType.DMA,
            pltpu.SemaphoreType.DMA,
        ],
    )
    def k1(sub_hbm, ent_t, inter, chunk, cent, cpos, eq, pq,
           win_a, win_b, tail, acc, aidx, sem_a, sem_b, sem_s):
        wid = lax.axis_index("s") * _NC + lax.axis_index("c")
        lo = (_NTF * wid) // _NW
        hi = (_NTF * (wid + 1)) // _NW
        is_last = wid == _NW - 1
        hi_m = jnp.where(is_last, _NT, hi)
        nwin = (hi - lo + _WC - 1) // _WC
        iota = lax.iota(jnp.int32, 16)
        dump = _B + wid * 16 + iota

        def init_aidx():
            def ib(j, _):
                aidx[pl.ds(j * 16, 16)] = dump
                return 0
            lax.fori_loop(0, _ACC // 16, ib, 0)

        init_aidx()

        def chunk_loop(k, cur):
            pltpu.sync_copy(sub_hbm.at[pl.ds(k * _CHI, _CHI)], chunk)

            def compress(j, cur2):
                e = chunk[pl.ds(j * 16, 16)]
                c = lax.shift_right_logical(e, 7)
                m = (c >= lo) & (c < hi_m)
                plsc.store_compressed(cent.at[pl.ds(cur2, 16)], e, mask=m)
                plsc.store_compressed(cpos.at[pl.ds(cur2, 16)],
                                      k * _CHI + j * 16 + iota, mask=m)
                return cur2 + plsc.all_reduce_population_count(m)[0]

            return lax.fori_loop(0, _CHI // 16, compress, cur)

        n_t = lax.fori_loop(0, _B // _CHI, chunk_loop, 0)
        npv = (n_t + 15) // 16

        def flush(cur):
            pltpu.async_copy(
                acc.at[pl.ds(0, _ACC), pl.ds(0, 128)], inter.at[aidx],
                sem_s).wait()
            init_aidx()
            return 0 * cur

        def emit(src, off, cnt, cur):
            e_loc = eq[pl.ds(0, 16)] - off
            pos = pq[pl.ds(0, 16)]
            m = iota < cnt
            rows = cur + iota

            def feat(i, _):
                for u in range(4):
                    fv = jnp.full((16,), 0, jnp.int32) + (i * 4 + u)
                    v = plsc.load_gather(src, [fv, e_loc], mask=m)
                    plsc.store_scatter(acc, [rows, fv], v, mask=m)
                return 0

            lax.fori_loop(0, _D // 4, feat, 0)
            plsc.store_scatter(aidx, [rows], jnp.where(m, pos, dump))
            cur = cur + 16
            return lax.cond(cur >= _ACC, flush, lambda c: c, cur)

        def drain(src, off, qn, cur):
            def full_emit(qc):
                q, c = qc
                c = emit(src, off, 16, c)
                ev = eq[pl.ds(16, 16)]
                pv = pq[pl.ds(16, 16)]
                eq[pl.ds(0, 16)] = ev
                pq[pl.ds(0, 16)] = pv
                return (q - 16, c)

            return lax.cond(qn >= 16, full_emit, lambda qc: qc, (qn, cur))

        def win_dma(w, buf, sem):
            start_c = jnp.minimum(lo + w * _WC, hi - _WC)
            return pltpu.make_async_copy(
                ent_t.at[pl.ds(0, _D), pl.ds(start_c * 128, _WE)], buf, sem)

        def process(w, src, cur):
            off = jnp.minimum(lo + w * _WC, hi - _WC) * 128

            def visit(j, qc):
                qn, c = qc
                e = cent[pl.ds(j * 16, 16)]
                pos = cpos[pl.ds(j * 16, 16)]
                valid = (j * 16 + iota) < n_t
                ct = lax.shift_right_logical(e, 7)
                wf = jnp.where(ct < hi,
                               jnp.minimum((ct - lo) // _WC, nwin - 1), nwin)
                m = (wf == w) & valid
                plsc.store_compressed(eq.at[pl.ds(qn, 16)], e, mask=m)
                plsc.store_compressed(pq.at[pl.ds(qn, 16)], pos, mask=m)
                qn = qn + plsc.all_reduce_population_count(m)[0]
                return drain(src, off, qn, c)

            qn, cur = lax.fori_loop(0, npv, visit, (0, cur))
            return lax.cond(qn > 0,
                            lambda c: emit(src, off, qn, c),
                            lambda c: c, cur)

        def one(w, cur):
            d = win_dma(w, win_a, sem_a)
            d.start()
            d.wait()
            return process(w, win_a, cur)

        cur = lax.fori_loop(0, nwin, one, 0)

        def tail_pass(c_in):
            off = _NTF * 128
            pltpu.sync_copy(
                ent_t.at[pl.ds(0, _D), pl.ds(off, _V - off)], tail)

            def visit_tail(j, qc):
                qn, c = qc
                e = cent[pl.ds(j * 16, 16)]
                pos = cpos[pl.ds(j * 16, 16)]
                valid = (j * 16 + iota) < n_t
                ct = lax.shift_right_logical(e, 7)
                m = (ct == _NT - 1) & valid
                plsc.store_compressed(eq.at[pl.ds(qn, 16)], e, mask=m)
                plsc.store_compressed(pq.at[pl.ds(qn, 16)], pos, mask=m)
                qn = qn + plsc.all_reduce_population_count(m)[0]
                return drain(tail, off, qn, c)

            qn, c = lax.fori_loop(0, npv, visit_tail, (0, c_in))
            return lax.cond(qn > 0,
                            lambda cc: emit(tail, off, qn, cc),
                            lambda cc: cc, c)

        cur = lax.cond(is_last, tail_pass, lambda c: c, cur)

        flush(cur)

    return k1


def _make_k2():
    _CH = _BPW // 4

    @functools.partial(
        pl.kernel,
        mesh=_mesh,
        compiler_params=_cp,
        out_type=jax.ShapeDtypeStruct((2 * _D, _B), jnp.float32),
        scratch_types=[
            pltpu.VMEM((_BPW,), jnp.int32),
            pltpu.VMEM((_CH, _STRIDE), jnp.float32),
            pltpu.VMEM((_D, _RV), jnp.float32),
            pltpu.VMEM((_D, _BPW), jnp.float32),
            pltpu.VMEM((_D, _BPW), jnp.float32),
        ],
    )
    def k2(rel_hbm, rel_t, inter, out, ridx_v, ibuf, rbuf, blk_e, blk_r):
        wid = lax.axis_index("s") * _NC + lax.axis_index("c")
        base = wid * _BPW
        pltpu.sync_copy(rel_hbm.at[pl.ds(base, _BPW)], ridx_v)
        pltpu.sync_copy(rel_t, rbuf)
        iota = lax.iota(jnp.int32, 16)
        for q in range(4):
            pltpu.sync_copy(inter.at[pl.ds(base + q * _CH, _CH)], ibuf)
            for f in range(_D):
                fv = jnp.full((16,), f, jnp.int32)

                def rowgrp(jj, _, q=q, f=f, fv=fv):
                    j = q * (_CH // 16) + jj
                    rows = jj * 16 + iota
                    r = ridx_v[pl.ds(j * 16, 16)]
                    ve = plsc.load_gather(ibuf, [rows, fv])
                    blk_e[f, pl.ds(j * 16, 16)] = ve
                    vr = plsc.load_gather(rbuf, [fv, r])
                    blk_r[f, pl.ds(j * 16, 16)] = vr
                    return 0

                lax.fori_loop(0, _CH // 16, rowgrp, 0)
        pltpu.sync_copy(blk_e, out.at[pl.ds(0, _D), pl.ds(base, _BPW)])
        pltpu.sync_copy(blk_r, out.at[pl.ds(_D, _D), pl.ds(base, _BPW)])

    return k2


_k1 = _make_k1()
_k2 = _make_k2()


def kernel(sub, rel, ent_emb, rel_emb):
    inter = _k1(sub.astype(jnp.int32), ent_emb.T)
    rel_t = jnp.pad(rel_emb.T, ((0, 0), (0, _RV - rel_emb.shape[0])))
    out_t = _k2(rel.astype(jnp.int32), rel_t, inter)
    return out_t.T


# speedup vs baseline: 1.1178x; 1.1178x over previous
import functools

import jax
import jax.numpy as jnp
from jax import lax
from jax.experimental import pallas as pl
from jax.experimental.pallas import tpu as pltpu
from jax.experimental.pallas import tpu_sc as plsc

_B = 16384
_D = 32
_V = 1000000
_RV = 1024
_NT = (_V + 127) // 128
_NTF = _V // 128
_WC = 8
_WE = _WC * 128
_CAP = _B + 32
_STRIDE = 128
_ACC = 144
_CHI = 2048

_info = plsc.get_sparse_core_info()
_NC, _NS = _info.num_cores, _info.num_subcores
_NW = _NC * _NS
_BPW = _B // _NW

_mesh = plsc.VectorSubcoreMesh(core_axis_name="c", subcore_axis_name="s")
_cp = pltpu.CompilerParams(use_tc_tiling_on_sc=True, needs_layout_passes=False)


def _make_k1():
    @functools.partial(
        pl.kernel,
        mesh=_mesh,
        compiler_params=_cp,
        out_type=jax.ShapeDtypeStruct((_B + 16 * _NW, 128), jnp.float32),
        scratch_types=[
            pltpu.VMEM((_CHI,), jnp.int32),
            pltpu.VMEM((_CAP,), jnp.int32),
            pltpu.VMEM((48,), jnp.int32),
            pltpu.VMEM((_D, _WE), jnp.float32),
            pltpu.VMEM((_D, _WE), jnp.float32),
            pltpu.VMEM((_D, 64), jnp.float32),
            pltpu.VMEM((_ACC, _STRIDE), jnp.float32),
            pltpu.VMEM((_ACC,), jnp.int32),
            pltpu.SemaphoreType.DMA,
            pltpu.SemaphoreType.DMA,
            pltpu.SemaphoreType.DMA,
        ],
    )
    def k1(sub_hbm, ent_t, inter, chunk, cpk, qpk,
           win_a, win_b, tail, acc, aidx, sem_a, sem_b, sem_s):
        wid = lax.axis_index("s") * _NC + lax.axis_index("c")
        lo = (_NTF * wid) // _NW
        hi = (_NTF * (wid + 1)) // _NW
        is_last = wid == _NW - 1
        hi_m = jnp.where(is_last, _NT, hi)
        nwin = (hi - lo + _WC - 1) // _WC
        iota = lax.iota(jnp.int32, 16)
        dump = _B + wid * 16 + iota

        def init_aidx():
            def ib(j, _):
                aidx[pl.ds(j * 16, 16)] = dump
                return 0
            lax.fori_loop(0, _ACC // 16, ib, 0)

        init_aidx()

        def chunk_loop(k, cur):
            pltpu.sync_copy(sub_hbm.at[pl.ds(k * _CHI, _CHI)], chunk)

            def compress(j, cur2):
                e = chunk[pl.ds(j * 16, 16)]
                c = lax.shift_right_logical(e, 7)
                m = (c >= lo) & (c < hi_m)
                pk = jnp.bitwise_or(
                    lax.shift_left(e - lo * 128, 14),
                    k * _CHI + j * 16 + iota)
                plsc.store_compressed(cpk.at[pl.ds(cur2, 16)], pk, mask=m)
                return cur2 + plsc.all_reduce_population_count(m)[0]

            return lax.fori_loop(0, _CHI // 16, compress, cur)

        n_t = lax.fori_loop(0, _B // _CHI, chunk_loop, 0)
        npv = (n_t + 15) // 16

        def flush(cur):
            pltpu.async_copy(
                acc.at[pl.ds(0, _ACC), pl.ds(0, 128)], inter.at[aidx],
                sem_s).wait()
            init_aidx()
            return 0 * cur

        def emit(src, off, cnt, cur):
            pk = qpk[pl.ds(0, 16)]
            e_loc = lax.shift_right_logical(pk, 14) + lo * 128 - off
            pos = jnp.bitwise_and(pk, 16383)
            m = iota < cnt
            rows = cur + iota

            def feat(i, _):
                for u in range(4):
                    fv = jnp.full((16,), 0, jnp.int32) + (i * 4 + u)
                    v = plsc.load_gather(src, [fv, e_loc], mask=m)
                    plsc.store_scatter(acc, [rows, fv], v, mask=m)
                return 0

            lax.fori_loop(0, _D // 4, feat, 0)
            plsc.store_scatter(aidx, [rows], jnp.where(m, pos, dump))
            cur = cur + 16
            return lax.cond(cur >= _ACC, flush, lambda c: c, cur)

        def drain(src, off, qn, cur):
            def full_emit(qc):
                q, c = qc
                c = emit(src, off, 16, c)
                pv = qpk[pl.ds(16, 16)]
                qpk[pl.ds(0, 16)] = pv
                return (q - 16, c)

            return lax.cond(qn >= 16, full_emit, lambda qc: qc, (qn, cur))

        def win_dma(w, buf, sem):
            start_c = jnp.minimum(lo + w * _WC, hi - _WC)
            return pltpu.make_async_copy(
                ent_t.at[pl.ds(0, _D), pl.ds(start_c * 128, _WE)], buf, sem)

        def process(w, src, cur):
            off = jnp.minimum(lo + w * _WC, hi - _WC) * 128

            def visit(j, qc):
                qn, c = qc
                pk = cpk[pl.ds(j * 16, 16)]
                valid = (j * 16 + iota) < n_t
                ct = lax.shift_right_logical(pk, 21) + lo
                wf = jnp.where(ct < hi,
                               jnp.minimum((ct - lo) // _WC, nwin - 1), nwin)
                m = (wf == w) & valid
                plsc.store_compressed(qpk.at[pl.ds(qn, 16)], pk, mask=m)
                qn = qn + plsc.all_reduce_population_count(m)[0]
                return drain(src, off, qn, c)

            qn, cur = lax.fori_loop(0, npv, visit, (0, cur))
            return lax.cond(qn > 0,
                            lambda c: emit(src, off, qn, c),
                            lambda c: c, cur)

        win_dma(0, win_a, sem_a).start()

        def pair(k, cur):
            w0 = 2 * k
            w1 = w0 + 1
            win_dma(w0, win_a, sem_a).wait()

            @pl.when(w1 < nwin)
            def _():
                win_dma(w1, win_b, sem_b).start()

            cur = process(w0, win_a, cur)

            def second(c):
                win_dma(w1, win_b, sem_b).wait()

                @pl.when(w1 + 1 < nwin)
                def _():
                    win_dma(w1 + 1, win_a, sem_a).start()

                return process(w1, win_b, c)

            return lax.cond(w1 < nwin, second, lambda c: c, cur)

        cur = lax.fori_loop(0, (nwin + 1) // 2, pair, 0)

        def tail_pass(c_in):
            off = _NTF * 128
            pltpu.sync_copy(
                ent_t.at[pl.ds(0, _D), pl.ds(off, _V - off)], tail)

            def visit_tail(j, qc):
                qn, c = qc
                pk = cpk[pl.ds(j * 16, 16)]
                valid = (j * 16 + iota) < n_t
                ct = lax.shift_right_logical(pk, 21) + lo
                m = (ct == _NT - 1) & valid
                plsc.store_compressed(qpk.at[pl.ds(qn, 16)], pk, mask=m)
                qn = qn + plsc.all_reduce_population_count(m)[0]
                return drain(tail, off, qn, c)

            qn, c = lax.fori_loop(0, npv, visit_tail, (0, c_in))
            return lax.cond(qn > 0,
                            lambda cc: emit(tail, off, qn, cc),
                            lambda cc: cc, c)

        cur = lax.cond(is_last, tail_pass, lambda c: c, cur)

        flush(cur)

    return k1


def _make_k2():
    _CH = _BPW // 4

    @functools.partial(
        pl.kernel,
        mesh=_mesh,
        compiler_params=_cp,
        out_type=jax.ShapeDtypeStruct((2 * _D, _B), jnp.float32),
        scratch_types=[
            pltpu.VMEM((_BPW,), jnp.int32),
            pltpu.VMEM((_CH, _STRIDE), jnp.float32),
            pltpu.VMEM((_D, _RV), jnp.float32),
            pltpu.VMEM((_D, _BPW), jnp.float32),
            pltpu.VMEM((_D, _BPW), jnp.float32),
        ],
    )
    def k2(rel_hbm, rel_t, inter, out, ridx_v, ibuf, rbuf, blk_e, blk_r):
        wid = lax.axis_index("s") * _NC + lax.axis_index("c")
        base = wid * _BPW
        pltpu.sync_copy(rel_hbm.at[pl.ds(base, _BPW)], ridx_v)
        pltpu.sync_copy(rel_t, rbuf)
        iota = lax.iota(jnp.int32, 16)
        for q in range(4):
            pltpu.sync_copy(inter.at[pl.ds(base + q * _CH, _CH)], ibuf)
            for f in range(_D):
                fv = jnp.full((16,), f, jnp.int32)

                def rowgrp(jj, _, q=q, f=f, fv=fv):
                    j = q * (_CH // 16) + jj
                    rows = jj * 16 + iota
                    r = ridx_v[pl.ds(j * 16, 16)]
                    ve = plsc.load_gather(ibuf, [rows, fv])
                    blk_e[f, pl.ds(j * 16, 16)] = ve
                    vr = plsc.load_gather(rbuf, [fv, r])
                    blk_r[f, pl.ds(j * 16, 16)] = vr
                    return 0

                lax.fori_loop(0, _CH // 16, rowgrp, 0)
        pltpu.sync_copy(blk_e, out.at[pl.ds(0, _D), pl.ds(base, _BPW)])
        pltpu.sync_copy(blk_r, out.at[pl.ds(_D, _D), pl.ds(base, _BPW)])

    return k2


_k1 = _make_k1()
_k2 = _make_k2()


def kernel(sub, rel, ent_emb, rel_emb):
    inter = _k1(sub.astype(jnp.int32), ent_emb.T)
    rel_t = jnp.pad(rel_emb.T, ((0, 0), (0, _RV - rel_emb.shape[0])))
    out_t = _k2(rel.astype(jnp.int32), rel_t, inter)
    return out_t.T
